# baseline (speedup 1.0000x reference)
import functools

import numpy as np
import jax
import jax.numpy as jnp
from jax import lax
from jax.experimental import pallas as pl
from jax.experimental.pallas import tpu as pltpu
from jax.experimental.pallas import tpu_sc as plsc

VOCAB = 100000
DIM = 1024
HALF = DIM // 2
BATCH = 4
SEQ = 2048
B = BATCH * SEQ

NC, NS = 2, 16
NW = NC * NS
CH = 16
MAXBUF = 6

SLICES = [(0, 1024), (1024, 512), (1536, 512)]

_sc_mesh = plsc.VectorSubcoreMesh(core_axis_name="c", subcore_axis_name="s")


def _gather_loop(table_hbm, idx_v, out_hbm, bufs, gsems, wsems, base, nch, nbuf):
    gs = [None] * nch
    ws = [None] * nch
    for j in range(min(nbuf, nch)):
        gs[j] = pltpu.async_copy(
            table_hbm.at[idx_v.at[pl.ds(j * CH, CH)]], bufs[j], gsems[j]
        )
    for j in range(nch):
        b = j % nbuf
        gs[j].wait()
        ws[j] = pltpu.async_copy(
            bufs[b], out_hbm.at[pl.ds(base + j * CH, CH)], wsems[b]
        )
        nxt = j + nbuf
        if nxt < nch:
            ws[j].wait()
            gs[nxt] = pltpu.async_copy(
                table_hbm.at[idx_v.at[pl.ds(nxt * CH, CH)]], bufs[b], gsems[b]
            )
    for j in range(max(0, nch - nbuf), nch):
        ws[j].wait()


def _make_sc_gather(pos_off, pos_len):
    rows = BATCH * pos_len
    bpw = rows // NW
    nch = bpw // CH
    nbuf = min(nch, MAXBUF)
    direct = bpw % 128 == 0
    wpb = pos_len // bpw

    scratch = [
        pltpu.VMEM((bpw,), jnp.int32) if direct
        else pltpu.VMEM((nch, CH), jnp.int32),
        [pltpu.VMEM((CH, DIM), jnp.float32) for _ in range(nbuf)],
        [pltpu.SemaphoreType.DMA for _ in range(nbuf)],
        [pltpu.SemaphoreType.DMA for _ in range(nbuf)],
    ]

    if direct:

        @functools.partial(
            pl.kernel,
            mesh=_sc_mesh,
            out_type=jax.ShapeDtypeStruct((rows, DIM), jnp.float32),
            scratch_types=scratch,
        )
        def _sc_gather(table_hbm, x_hbm, out_hbm, idx_v, bufs, gsems, wsems):
            wid = lax.axis_index("s") * NC + lax.axis_index("c")
            brow = wid // wpb
            col0 = (wid % wpb) * bpw + pos_off
            pltpu.sync_copy(x_hbm.at[brow, pl.ds(col0, bpw)], idx_v)
            _gather_loop(table_hbm, idx_v, out_hbm, bufs, gsems, wsems,
                         wid * bpw, nch, nbuf)

        return lambda W, x: _sc_gather(W, x)

    @functools.partial(
        pl.kernel,
        mesh=_sc_mesh,
        out_type=jax.ShapeDtypeStruct((rows, DIM), jnp.float32),
        scratch_types=scratch,
    )
    def _sc_gather_op(table_hbm, idx_hbm, out_hbm, idx_v, bufs, gsems, wsems):
        wid = lax.axis_index("s") * NC + lax.axis_index("c")
        pltpu.sync_copy(idx_hbm.at[wid], idx_v)
        _gather_loop_2d(table_hbm, idx_v, out_hbm, bufs, gsems, wsems,
                        wid * bpw, nch, nbuf)

    def _call(W, x):
        idx = x[:, pos_off : pos_off + pos_len].reshape(NW, nch, CH)
        return _sc_gather_op(W, idx)

    return _call


def _gather_loop_2d(table_hbm, idx_v, out_hbm, bufs, gsems, wsems, base, nch, nbuf):
    gs = [None] * nch
    ws = [None] * nch
    for j in range(min(nbuf, nch)):
        gs[j] = pltpu.async_copy(table_hbm.at[idx_v.at[j]], bufs[j], gsems[j])
    for j in range(nch):
        b = j % nbuf
        gs[j].wait()
        ws[j] = pltpu.async_copy(
            bufs[b], out_hbm.at[pl.ds(base + j * CH, CH)], wsems[b]
        )
        nxt = j + nbuf
        if nxt < nch:
            ws[j].wait()
            gs[nxt] = pltpu.async_copy(
                table_hbm.at[idx_v.at[nxt]], bufs[b], gsems[b]
            )
    for j in range(max(0, nch - nbuf), nch):
        ws[j].wait()


_SC_GATHER = [_make_sc_gather(o, p) for o, p in SLICES]


def _rope_tables():
    fi = np.arange(HALF, dtype=np.float32)
    freqs = (1.0 / (10000.0 ** (fi / DIM))).astype(np.float32)
    pos = np.arange(SEQ, dtype=np.float32)
    angles = pos[:, None] * freqs[None, :]
    return np.sin(angles), np.cos(angles)


_SIN_NP, _COS_NP = _rope_tables()


def _rope_first_body(e_ref, s_ref, c_ref, o_ref):
    xe = e_ref[:, :HALF]
    xo = e_ref[:, HALF:]
    s = s_ref[...].astype(jnp.float32)
    c = c_ref[...].astype(jnp.float32)
    o_ref[:, :HALF] = xe * c - xo * s
    o_ref[:, HALF:] = xe * s + xo * c


def _rope_chain_body(e_ref, s_ref, c_ref, prev_ref, o_ref):
    del prev_ref
    _rope_first_body(e_ref, s_ref, c_ref, o_ref)


def _make_rope(slice_idx):
    pos_off, pos_len = SLICES[slice_idx]
    tb = pos_off // pos_len
    ob = SEQ // pos_len
    in_specs = [
        pl.BlockSpec((pos_len, DIM), lambda b: (b, 0)),
        pl.BlockSpec((pos_len, HALF), lambda b, t=tb: (t, 0)),
        pl.BlockSpec((pos_len, HALF), lambda b, t=tb: (t, 0)),
    ]
    body = _rope_first_body
    aliases = {}
    if slice_idx > 0:
        in_specs.append(pl.BlockSpec(memory_space=pl.MemorySpace.ANY))
        body = _rope_chain_body
        aliases = {3: 0}
    return pl.pallas_call(
        body,
        grid=(BATCH,),
        in_specs=in_specs,
        out_specs=pl.BlockSpec(
            (pos_len, DIM), lambda b, t=tb, n=ob: (b * n + t, 0)
        ),
        out_shape=jax.ShapeDtypeStruct((B, DIM), jnp.float32),
        input_output_aliases=aliases,
        name=f"rope_slice_{slice_idx}",
    )


_ROPE = [_make_rope(s) for s in range(len(SLICES))]


def kernel(x, W):
    sin_t = jnp.asarray(_SIN_NP, dtype=jnp.bfloat16)
    cos_t = jnp.asarray(_COS_NP, dtype=jnp.bfloat16)
    embs = [g(W, x) for g in _SC_GATHER]
    out = _ROPE[0](embs[0], sin_t, cos_t)
    for s in range(1, len(SLICES)):
        out = _ROPE[s](embs[s], sin_t, cos_t, out)
    return out.reshape(BATCH, SEQ, DIM)
